# baseline (speedup 1.0000x reference)
import functools

import jax
import jax.numpy as jnp
from jax import lax
from jax.experimental import pallas as pl
from jax.experimental.pallas import tpu as pltpu
from jax.experimental.pallas import tpu_sc as plsc

N = 10000
DH = 64
G = 64
NP = 10016
RPS = NP // 16
CHUNK = 128
CH = 80
CH2 = 160
HC = DH // 2
NW = 32
EPAD = NW * CH * CHUNK
DEGW = 8
NBUF = 8
EPS = 1e-5

def _deg_body(dst_hbm, ones_hbm, z_hbm, out_hbm, idx_v, ones_v, acc):
    c = lax.axis_index("c")
    s = lax.axis_index("s")
    w = s * 2 + c
    pltpu.sync_copy(z_hbm, acc.at[pl.ds(s * RPS, RPS)])
    pltpu.sync_copy(ones_hbm, ones_v)
    pltpu.sync_copy(dst_hbm.at[w], idx_v)
    plsc.subcore_barrier()

    def body(j, carry):
        pltpu.sync_copy(ones_v, acc.at[idx_v.at[j]], add=True)
        return carry

    lax.fori_loop(0, CH, body, 0)
    plsc.subcore_barrier()
    pltpu.sync_copy(acc.at[pl.ds(s * RPS, RPS)], out_hbm.at[c, pl.ds(s * RPS, RPS)])


def _scatter_body(h_hbm, src_hbm, dst_hbm, z_hbm, out_hbm, isrc, idst, buf, acc,
                  h_s, *sems):
    c = lax.axis_index("c")
    s = lax.axis_index("s")
    col = c * HC
    pltpu.sync_copy(z_hbm, acc.at[pl.ds(s * RPS, RPS)])
    pltpu.sync_copy(src_hbm.at[s], isrc)
    pltpu.sync_copy(dst_hbm.at[s], idst)
    pltpu.sync_copy(h_hbm.at[pl.ds(s * (N // 16), N // 16), pl.ds(col, HC)],
                    h_s.at[pl.ds(s * (N // 16), N // 16)])
    plsc.subcore_barrier()

    gsems = sems[:NBUF]
    ssems = sems[NBUF:]
    for b in range(NBUF):
        pltpu.async_copy(h_s.at[isrc.at[b]], buf.at[b], gsems[b])

    def body(i, carry):
        for b in range(NBUF):
            j = i * NBUF + b
            bp = (b - 1) % NBUF
            pltpu.make_async_copy(h_s.at[isrc.at[j]], buf.at[b], gsems[b]).wait()
            pltpu.async_copy(buf.at[b], acc.at[idst.at[j]], ssems[b], add=True)
            regather = j + NBUF - 1 < CH2
            if b == 0:
                @pl.when(i > 0)
                def _():
                    pltpu.make_async_copy(buf.at[bp], acc.at[idst.at[0]],
                                          ssems[bp]).wait()

                @pl.when((i > 0) & regather)
                def _():
                    pltpu.async_copy(h_s.at[isrc.at[j + NBUF - 1]],
                                     buf.at[bp], gsems[bp])
            else:
                pltpu.make_async_copy(buf.at[bp], acc.at[idst.at[0]],
                                      ssems[bp]).wait()

                @pl.when(regather)
                def _():
                    pltpu.async_copy(h_s.at[isrc.at[j + NBUF - 1]],
                                     buf.at[bp], gsems[bp])
        return carry

    lax.fori_loop(0, CH2 // NBUF, body, 0)
    pltpu.make_async_copy(buf.at[NBUF - 1], acc.at[idst.at[0]],
                          ssems[NBUF - 1]).wait()
    plsc.subcore_barrier()
    pltpu.sync_copy(acc.at[pl.ds(s * RPS, RPS)],
                    out_hbm.at[pl.ds(s * RPS, RPS), pl.ds(col, HC)])


@functools.cache
def _sc_kernels():
    mesh = plsc.VectorSubcoreMesh(
        core_axis_name="c", subcore_axis_name="s", num_cores=2, num_subcores=16)
    params = pltpu.CompilerParams(use_tc_tiling_on_sc=False)
    deg_k = pl.kernel(
        _deg_body,
        out_type=jax.ShapeDtypeStruct((2, NP, DEGW), jnp.float32),
        mesh=mesh,
        compiler_params=params,
        scratch_types=[
            pltpu.VMEM((CH, CHUNK), jnp.int32),
            pltpu.VMEM((CHUNK, DEGW), jnp.float32),
            pltpu.VMEM_SHARED((NP, DEGW), jnp.float32),
        ],
    )
    scat_k = pl.kernel(
        _scatter_body,
        out_type=jax.ShapeDtypeStruct((NP, DH), jnp.float32),
        mesh=mesh,
        compiler_params=params,
        scratch_types=[
            pltpu.VMEM((CH2, CHUNK), jnp.int32),
            pltpu.VMEM((CH2, CHUNK), jnp.int32),
            pltpu.VMEM((NBUF, CHUNK, HC), jnp.float32),
            pltpu.VMEM_SHARED((NP, HC), jnp.float32),
            pltpu.VMEM_SHARED((N, HC), jnp.float32),
        ] + [pltpu.SemaphoreType.DMA] * (2 * NBUF),
    )
    return deg_k, scat_k


def _dinv_from(deg_all):
    d = deg_all[0, :N, 0:1] + deg_all[1, :N, 0:1] + 1.0
    return 1.0 / jnp.sqrt(d)


def _tc_first_body(deg_ref, x_ref, w_ref, out_ref):
    dinv = _dinv_from(deg_ref[...])
    mm = jnp.dot(x_ref[...], w_ref[...], preferred_element_type=jnp.float32)
    out_ref[...] = dinv * mm


def _post_conv(deg_ref, acc_ref, hp_ref, b_ref, g_ref, be_ref):
    dinv = _dinv_from(deg_ref[...])
    z = dinv * (acc_ref[...][:N, :] + hp_ref[...]) + b_ref[...]
    z = jnp.maximum(z, 0.0)
    m = jnp.mean(z, axis=0, keepdims=True)
    zc = z - m
    v = jnp.mean(zc * zc, axis=0, keepdims=True)
    z = zc / jnp.sqrt(v + EPS) * g_ref[...] + be_ref[...]
    return dinv, z


def _tc_mid_body(deg_ref, acc_ref, hp_ref, b_ref, g_ref, be_ref, w_ref, out_ref):
    dinv, z = _post_conv(deg_ref, acc_ref, hp_ref, b_ref, g_ref, be_ref)
    out_ref[...] = dinv * jnp.dot(z, w_ref[...], preferred_element_type=jnp.float32)


def _tc_final_body(deg_ref, acc_ref, hp_ref, b_ref, g_ref, be_ref, bat_ref, out_ref):
    _, z = _post_conv(deg_ref, acc_ref, hp_ref, b_ref, g_ref, be_ref)
    gid = lax.broadcasted_iota(jnp.int32, (G, N), 0)
    oh = jnp.where(gid == bat_ref[...], 1.0, 0.0)
    sums = jnp.dot(oh, z, preferred_element_type=jnp.float32)
    ones_col = jnp.full((N, 1), 1.0, jnp.float32)
    cnt = jnp.dot(oh, ones_col, preferred_element_type=jnp.float32)
    out_ref[...] = sums / jnp.maximum(cnt, 1.0)


_tc_first = pl.pallas_call(
    _tc_first_body, out_shape=jax.ShapeDtypeStruct((N, DH), jnp.float32))
_tc_mid = pl.pallas_call(
    _tc_mid_body, out_shape=jax.ShapeDtypeStruct((N, DH), jnp.float32))
_tc_final = pl.pallas_call(
    _tc_final_body, out_shape=jax.ShapeDtypeStruct((G, DH), jnp.float32))


def kernel(x, edge_index, batch, W1, b1, g1, be1, W2, b2, g2, be2, W3, b3, g3, be3):
    src = edge_index[0]
    dst = edge_index[1]
    pad = EPAD - src.shape[0]
    spad = jnp.concatenate([src, jnp.zeros((pad,), jnp.int32)])
    dpad = jnp.concatenate([dst, jnp.full((pad,), N, jnp.int32)])
    dstp = dpad.reshape(NW, CH, CHUNK)
    srcp2 = spad.reshape(16, CH2, CHUNK)
    dstp2 = dpad.reshape(16, CH2, CHUNK)
    ones16 = jnp.zeros((CHUNK, DEGW), jnp.float32).at[:, 0].set(1.0)
    z16 = jnp.zeros((RPS, DEGW), jnp.float32)
    z32 = jnp.zeros((RPS, HC), jnp.float32)
    b1r, g1r, be1r = b1.reshape(1, DH), g1.reshape(1, DH), be1.reshape(1, DH)
    b2r, g2r, be2r = b2.reshape(1, DH), g2.reshape(1, DH), be2.reshape(1, DH)
    b3r, g3r, be3r = b3.reshape(1, DH), g3.reshape(1, DH), be3.reshape(1, DH)
    batr = batch.reshape(1, N)

    _deg_kernel, _scatter_kernel = _sc_kernels()
    degp = _deg_kernel(dstp, ones16, z16)
    h1p = _tc_first(degp, x, W1)
    acc1 = _scatter_kernel(h1p, srcp2, dstp2, z32)
    h2p = _tc_mid(degp, acc1, h1p, b1r, g1r, be1r, W2)
    acc2 = _scatter_kernel(h2p, srcp2, dstp2, z32)
    h3p = _tc_mid(degp, acc2, h2p, b2r, g2r, be2r, W3)
    acc3 = _scatter_kernel(h3p, srcp2, dstp2, z32)
    out = _tc_final(degp, acc3, h3p, b3r, g3r, be3r, batr)
    return out


# speedup vs baseline: 1.0061x; 1.0061x over previous
import functools

import jax
import jax.numpy as jnp
from jax import lax
from jax.experimental import pallas as pl
from jax.experimental.pallas import tpu as pltpu
from jax.experimental.pallas import tpu_sc as plsc

N = 10000
DH = 64
G = 64
NP = 10016
RPS = NP // 16
CHUNK = 128
CH = 80
CH2 = 160
HC = DH // 2
NW = 32
EPAD = NW * CH * CHUNK
DEGW = 8
NBUF = 4
EPS = 1e-5

def _deg_body(dst_hbm, ones_hbm, z_hbm, out_hbm, idx_v, ones_v, acc):
    c = lax.axis_index("c")
    s = lax.axis_index("s")
    w = s * 2 + c
    pltpu.sync_copy(z_hbm, acc.at[pl.ds(s * RPS, RPS)])
    pltpu.sync_copy(ones_hbm, ones_v)
    pltpu.sync_copy(dst_hbm.at[w], idx_v)
    plsc.subcore_barrier()

    def body(j, carry):
        pltpu.sync_copy(ones_v, acc.at[idx_v.at[j]], add=True)
        return carry

    lax.fori_loop(0, CH, body, 0)
    plsc.subcore_barrier()
    pltpu.sync_copy(acc.at[pl.ds(s * RPS, RPS)], out_hbm.at[c, pl.ds(s * RPS, RPS)])


def _scatter_body(h_hbm, src_hbm, dst_hbm, z_hbm, out_hbm, isrc, idst, buf, acc,
                  h_s, *sems):
    c = lax.axis_index("c")
    s = lax.axis_index("s")
    col = c * HC
    pltpu.sync_copy(z_hbm, acc.at[pl.ds(s * RPS, RPS)])
    pltpu.sync_copy(src_hbm.at[s], isrc)
    pltpu.sync_copy(dst_hbm.at[s], idst)
    pltpu.sync_copy(h_hbm.at[pl.ds(s * (N // 16), N // 16), pl.ds(col, HC)],
                    h_s.at[pl.ds(s * (N // 16), N // 16)])
    plsc.subcore_barrier()

    gsems = sems[:NBUF]
    ssems = sems[NBUF:]
    for b in range(NBUF):
        pltpu.async_copy(h_s.at[isrc.at[b]], buf.at[b], gsems[b])

    def body(i, carry):
        for b in range(NBUF):
            j = i * NBUF + b
            bp = (b - 1) % NBUF
            pltpu.make_async_copy(h_s.at[isrc.at[j]], buf.at[b], gsems[b]).wait()
            pltpu.async_copy(buf.at[b], acc.at[idst.at[j]], ssems[b], add=True)
            regather = j + NBUF - 1 < CH2
            if b == 0:
                @pl.when(i > 0)
                def _():
                    pltpu.make_async_copy(buf.at[bp], acc.at[idst.at[0]],
                                          ssems[bp]).wait()

                @pl.when((i > 0) & regather)
                def _():
                    pltpu.async_copy(h_s.at[isrc.at[j + NBUF - 1]],
                                     buf.at[bp], gsems[bp])
            else:
                pltpu.make_async_copy(buf.at[bp], acc.at[idst.at[0]],
                                      ssems[bp]).wait()

                @pl.when(regather)
                def _():
                    pltpu.async_copy(h_s.at[isrc.at[j + NBUF - 1]],
                                     buf.at[bp], gsems[bp])
        return carry

    lax.fori_loop(0, CH2 // NBUF, body, 0)
    pltpu.make_async_copy(buf.at[NBUF - 1], acc.at[idst.at[0]],
                          ssems[NBUF - 1]).wait()
    plsc.subcore_barrier()
    pltpu.sync_copy(acc.at[pl.ds(s * RPS, RPS)],
                    out_hbm.at[pl.ds(s * RPS, RPS), pl.ds(col, HC)])


@functools.cache
def _sc_kernels():
    mesh = plsc.VectorSubcoreMesh(
        core_axis_name="c", subcore_axis_name="s", num_cores=2, num_subcores=16)
    params = pltpu.CompilerParams(use_tc_tiling_on_sc=False)
    deg_k = pl.kernel(
        _deg_body,
        out_type=jax.ShapeDtypeStruct((2, NP, DEGW), jnp.float32),
        mesh=mesh,
        compiler_params=params,
        scratch_types=[
            pltpu.VMEM((CH, CHUNK), jnp.int32),
            pltpu.VMEM((CHUNK, DEGW), jnp.float32),
            pltpu.VMEM_SHARED((NP, DEGW), jnp.float32),
        ],
    )
    scat_k = pl.kernel(
        _scatter_body,
        out_type=jax.ShapeDtypeStruct((NP, DH), jnp.float32),
        mesh=mesh,
        compiler_params=params,
        scratch_types=[
            pltpu.VMEM((CH2, CHUNK), jnp.int32),
            pltpu.VMEM((CH2, CHUNK), jnp.int32),
            pltpu.VMEM((NBUF, CHUNK, HC), jnp.float32),
            pltpu.VMEM_SHARED((NP, HC), jnp.float32),
            pltpu.VMEM_SHARED((N, HC), jnp.float32),
        ] + [pltpu.SemaphoreType.DMA] * (2 * NBUF),
    )
    return deg_k, scat_k


def _dinv_from(deg_all):
    d = deg_all[0, :N, 0:1] + deg_all[1, :N, 0:1] + 1.0
    return 1.0 / jnp.sqrt(d)


def _tc_first_body(deg_ref, x_ref, w_ref, out_ref):
    dinv = _dinv_from(deg_ref[...])
    mm = jnp.dot(x_ref[...], w_ref[...], preferred_element_type=jnp.float32)
    out_ref[...] = dinv * mm


def _post_conv(deg_ref, acc_ref, hp_ref, b_ref, g_ref, be_ref):
    dinv = _dinv_from(deg_ref[...])
    z = dinv * (acc_ref[...][:N, :] + hp_ref[...]) + b_ref[...]
    z = jnp.maximum(z, 0.0)
    m = jnp.mean(z, axis=0, keepdims=True)
    zc = z - m
    v = jnp.mean(zc * zc, axis=0, keepdims=True)
    z = zc / jnp.sqrt(v + EPS) * g_ref[...] + be_ref[...]
    return dinv, z


def _tc_mid_body(deg_ref, acc_ref, hp_ref, b_ref, g_ref, be_ref, w_ref, out_ref):
    dinv, z = _post_conv(deg_ref, acc_ref, hp_ref, b_ref, g_ref, be_ref)
    out_ref[...] = dinv * jnp.dot(z, w_ref[...], preferred_element_type=jnp.float32)


def _tc_final_body(deg_ref, acc_ref, hp_ref, b_ref, g_ref, be_ref, bat_ref, out_ref):
    _, z = _post_conv(deg_ref, acc_ref, hp_ref, b_ref, g_ref, be_ref)
    gid = lax.broadcasted_iota(jnp.int32, (G, N), 0)
    oh = jnp.where(gid == bat_ref[...], 1.0, 0.0)
    sums = jnp.dot(oh, z, preferred_element_type=jnp.float32)
    ones_col = jnp.full((N, 1), 1.0, jnp.float32)
    cnt = jnp.dot(oh, ones_col, preferred_element_type=jnp.float32)
    out_ref[...] = sums / jnp.maximum(cnt, 1.0)


_tc_first = pl.pallas_call(
    _tc_first_body, out_shape=jax.ShapeDtypeStruct((N, DH), jnp.float32))
_tc_mid = pl.pallas_call(
    _tc_mid_body, out_shape=jax.ShapeDtypeStruct((N, DH), jnp.float32))
_tc_final = pl.pallas_call(
    _tc_final_body, out_shape=jax.ShapeDtypeStruct((G, DH), jnp.float32))


def kernel(x, edge_index, batch, W1, b1, g1, be1, W2, b2, g2, be2, W3, b3, g3, be3):
    src = edge_index[0]
    dst = edge_index[1]
    pad = EPAD - src.shape[0]
    spad = jnp.concatenate([src, jnp.zeros((pad,), jnp.int32)])
    dpad = jnp.concatenate([dst, jnp.full((pad,), N, jnp.int32)])
    dstp = dpad.reshape(NW, CH, CHUNK)
    srcp2 = spad.reshape(16, CH2, CHUNK)
    dstp2 = dpad.reshape(16, CH2, CHUNK)
    ones16 = jnp.zeros((CHUNK, DEGW), jnp.float32).at[:, 0].set(1.0)
    z16 = jnp.zeros((RPS, DEGW), jnp.float32)
    z32 = jnp.zeros((RPS, HC), jnp.float32)
    b1r, g1r, be1r = b1.reshape(1, DH), g1.reshape(1, DH), be1.reshape(1, DH)
    b2r, g2r, be2r = b2.reshape(1, DH), g2.reshape(1, DH), be2.reshape(1, DH)
    b3r, g3r, be3r = b3.reshape(1, DH), g3.reshape(1, DH), be3.reshape(1, DH)
    batr = batch.reshape(1, N)

    _deg_kernel, _scatter_kernel = _sc_kernels()
    degp = _deg_kernel(dstp, ones16, z16)
    h1p = _tc_first(degp, x, W1)
    acc1 = _scatter_kernel(h1p, srcp2, dstp2, z32)
    h2p = _tc_mid(degp, acc1, h1p, b1r, g1r, be1r, W2)
    acc2 = _scatter_kernel(h2p, srcp2, dstp2, z32)
    h3p = _tc_mid(degp, acc2, h2p, b2r, g2r, be2r, W3)
    acc3 = _scatter_kernel(h3p, srcp2, dstp2, z32)
    out = _tc_final(degp, acc3, h3p, b3r, g3r, be3r, batr)
    return out


# speedup vs baseline: 1.0065x; 1.0004x over previous
import functools

import jax
import jax.numpy as jnp
from jax import lax
from jax.experimental import pallas as pl
from jax.experimental.pallas import tpu as pltpu
from jax.experimental.pallas import tpu_sc as plsc

N = 10000
DH = 64
G = 64
NP = 10016
RPS = NP // 16
CHUNK = 128
CH = 80
CH2 = 160
HC = DH // 2
NW = 32
EPAD = NW * CH * CHUNK
DEGW = 8
NBUF = 4
EPS = 1e-5

def _deg_body(dst_hbm, ones_hbm, z_hbm, out_hbm, idx_v, ones_v, acc, *sems):
    c = lax.axis_index("c")
    s = lax.axis_index("s")
    w = s * 2 + c
    pltpu.sync_copy(z_hbm, acc.at[pl.ds(s * RPS, RPS)])
    pltpu.sync_copy(ones_hbm, ones_v)
    pltpu.sync_copy(dst_hbm.at[w], idx_v)
    plsc.subcore_barrier()

    for b in range(4):
        pltpu.async_copy(ones_v, acc.at[idx_v.at[b]], sems[b], add=True)

    def body(i, carry):
        for b in range(4):
            j = (i + 1) * 4 + b
            pltpu.make_async_copy(ones_v, acc.at[idx_v.at[0]], sems[b]).wait()
            pltpu.async_copy(ones_v, acc.at[idx_v.at[j]], sems[b], add=True)
        return carry

    lax.fori_loop(0, CH // 4 - 1, body, 0)
    for b in range(4):
        pltpu.make_async_copy(ones_v, acc.at[idx_v.at[0]], sems[b]).wait()
    plsc.subcore_barrier()
    pltpu.sync_copy(acc.at[pl.ds(s * RPS, RPS)], out_hbm.at[c, pl.ds(s * RPS, RPS)])


def _scatter_body(h_hbm, src_hbm, dst_hbm, z_hbm, out_hbm, isrc, idst, buf, acc,
                  h_s, *sems):
    c = lax.axis_index("c")
    s = lax.axis_index("s")
    col = c * HC
    pltpu.sync_copy(z_hbm, acc.at[pl.ds(s * RPS, RPS)])
    pltpu.sync_copy(src_hbm.at[s], isrc)
    pltpu.sync_copy(dst_hbm.at[s], idst)
    pltpu.sync_copy(h_hbm.at[pl.ds(s * (N // 16), N // 16), pl.ds(col, HC)],
                    h_s.at[pl.ds(s * (N // 16), N // 16)])
    plsc.subcore_barrier()

    gsems = sems[:NBUF]
    ssems = sems[NBUF:]
    for b in range(NBUF):
        pltpu.async_copy(h_s.at[isrc.at[b]], buf.at[b], gsems[b])

    def body(i, carry):
        for b in range(NBUF):
            j = i * NBUF + b
            bp = (b - 1) % NBUF
            pltpu.make_async_copy(h_s.at[isrc.at[j]], buf.at[b], gsems[b]).wait()
            pltpu.async_copy(buf.at[b], acc.at[idst.at[j]], ssems[b], add=True)
            regather = j + NBUF - 1 < CH2
            if b == 0:
                @pl.when(i > 0)
                def _():
                    pltpu.make_async_copy(buf.at[bp], acc.at[idst.at[0]],
                                          ssems[bp]).wait()

                @pl.when((i > 0) & regather)
                def _():
                    pltpu.async_copy(h_s.at[isrc.at[j + NBUF - 1]],
                                     buf.at[bp], gsems[bp])
            else:
                pltpu.make_async_copy(buf.at[bp], acc.at[idst.at[0]],
                                      ssems[bp]).wait()

                @pl.when(regather)
                def _():
                    pltpu.async_copy(h_s.at[isrc.at[j + NBUF - 1]],
                                     buf.at[bp], gsems[bp])
        return carry

    lax.fori_loop(0, CH2 // NBUF, body, 0)
    pltpu.make_async_copy(buf.at[NBUF - 1], acc.at[idst.at[0]],
                          ssems[NBUF - 1]).wait()
    plsc.subcore_barrier()
    pltpu.sync_copy(acc.at[pl.ds(s * RPS, RPS)],
                    out_hbm.at[pl.ds(s * RPS, RPS), pl.ds(col, HC)])


@functools.cache
def _sc_kernels():
    mesh = plsc.VectorSubcoreMesh(
        core_axis_name="c", subcore_axis_name="s", num_cores=2, num_subcores=16)
    params = pltpu.CompilerParams(use_tc_tiling_on_sc=False)
    deg_k = pl.kernel(
        _deg_body,
        out_type=jax.ShapeDtypeStruct((2, NP, DEGW), jnp.float32),
        mesh=mesh,
        compiler_params=params,
        scratch_types=[
            pltpu.VMEM((CH, CHUNK), jnp.int32),
            pltpu.VMEM((CHUNK, DEGW), jnp.float32),
            pltpu.VMEM_SHARED((NP, DEGW), jnp.float32),
        ] + [pltpu.SemaphoreType.DMA] * 4,
    )
    scat_k = pl.kernel(
        _scatter_body,
        out_type=jax.ShapeDtypeStruct((NP, DH), jnp.float32),
        mesh=mesh,
        compiler_params=params,
        scratch_types=[
            pltpu.VMEM((CH2, CHUNK), jnp.int32),
            pltpu.VMEM((CH2, CHUNK), jnp.int32),
            pltpu.VMEM((NBUF, CHUNK, HC), jnp.float32),
            pltpu.VMEM_SHARED((NP, HC), jnp.float32),
            pltpu.VMEM_SHARED((N, HC), jnp.float32),
        ] + [pltpu.SemaphoreType.DMA] * (2 * NBUF),
    )
    return deg_k, scat_k


def _dinv_from(deg_all):
    d = deg_all[0, :N, 0:1] + deg_all[1, :N, 0:1] + 1.0
    return 1.0 / jnp.sqrt(d)


def _tc_first_body(deg_ref, x_ref, w_ref, out_ref):
    dinv = _dinv_from(deg_ref[...])
    mm = jnp.dot(x_ref[...], w_ref[...], preferred_element_type=jnp.float32)
    out_ref[...] = dinv * mm


def _post_conv(deg_ref, acc_ref, hp_ref, b_ref, g_ref, be_ref):
    dinv = _dinv_from(deg_ref[...])
    z = dinv * (acc_ref[...][:N, :] + hp_ref[...]) + b_ref[...]
    z = jnp.maximum(z, 0.0)
    m = jnp.mean(z, axis=0, keepdims=True)
    zc = z - m
    v = jnp.mean(zc * zc, axis=0, keepdims=True)
    z = zc / jnp.sqrt(v + EPS) * g_ref[...] + be_ref[...]
    return dinv, z


def _tc_mid_body(deg_ref, acc_ref, hp_ref, b_ref, g_ref, be_ref, w_ref, out_ref):
    dinv, z = _post_conv(deg_ref, acc_ref, hp_ref, b_ref, g_ref, be_ref)
    out_ref[...] = dinv * jnp.dot(z, w_ref[...], preferred_element_type=jnp.float32)


def _tc_final_body(deg_ref, acc_ref, hp_ref, b_ref, g_ref, be_ref, bat_ref, out_ref):
    _, z = _post_conv(deg_ref, acc_ref, hp_ref, b_ref, g_ref, be_ref)
    gid = lax.broadcasted_iota(jnp.int32, (G, N), 0)
    oh = jnp.where(gid == bat_ref[...], 1.0, 0.0)
    sums = jnp.dot(oh, z, preferred_element_type=jnp.float32)
    ones_col = jnp.full((N, 1), 1.0, jnp.float32)
    cnt = jnp.dot(oh, ones_col, preferred_element_type=jnp.float32)
    out_ref[...] = sums / jnp.maximum(cnt, 1.0)


_tc_first = pl.pallas_call(
    _tc_first_body, out_shape=jax.ShapeDtypeStruct((N, DH), jnp.float32))
_tc_mid = pl.pallas_call(
    _tc_mid_body, out_shape=jax.ShapeDtypeStruct((N, DH), jnp.float32))
_tc_final = pl.pallas_call(
    _tc_final_body, out_shape=jax.ShapeDtypeStruct((G, DH), jnp.float32))


def kernel(x, edge_index, batch, W1, b1, g1, be1, W2, b2, g2, be2, W3, b3, g3, be3):
    src = edge_index[0]
    dst = edge_index[1]
    pad = EPAD - src.shape[0]
    spad = jnp.concatenate([src, jnp.zeros((pad,), jnp.int32)])
    dpad = jnp.concatenate([dst, jnp.full((pad,), N, jnp.int32)])
    dstp = dpad.reshape(NW, CH, CHUNK)
    srcp2 = spad.reshape(16, CH2, CHUNK)
    dstp2 = dpad.reshape(16, CH2, CHUNK)
    ones16 = jnp.zeros((CHUNK, DEGW), jnp.float32).at[:, 0].set(1.0)
    z16 = jnp.zeros((RPS, DEGW), jnp.float32)
    z32 = jnp.zeros((RPS, HC), jnp.float32)
    b1r, g1r, be1r = b1.reshape(1, DH), g1.reshape(1, DH), be1.reshape(1, DH)
    b2r, g2r, be2r = b2.reshape(1, DH), g2.reshape(1, DH), be2.reshape(1, DH)
    b3r, g3r, be3r = b3.reshape(1, DH), g3.reshape(1, DH), be3.reshape(1, DH)
    batr = batch.reshape(1, N)

    _deg_kernel, _scatter_kernel = _sc_kernels()
    degp = _deg_kernel(dstp, ones16, z16)
    h1p = _tc_first(degp, x, W1)
    acc1 = _scatter_kernel(h1p, srcp2, dstp2, z32)
    h2p = _tc_mid(degp, acc1, h1p, b1r, g1r, be1r, W2)
    acc2 = _scatter_kernel(h2p, srcp2, dstp2, z32)
    h3p = _tc_mid(degp, acc2, h2p, b2r, g2r, be2r, W3)
    acc3 = _scatter_kernel(h3p, srcp2, dstp2, z32)
    out = _tc_final(degp, acc3, h3p, b3r, g3r, be3r, batr)
    return out
